# Initial kernel scaffold; baseline (speedup 1.0000x reference)
import jax
import jax.numpy as jnp
from jax.experimental import pallas as pl


def kernel(x, edge_index, batch, params):
    raise NotImplementedError("write your pallas kernel here")


# speedup vs baseline: 6.3069x; 6.3069x over previous
import functools

import jax
import jax.numpy as jnp
from jax import lax
from jax.experimental import pallas as pl
from jax.experimental.pallas import tpu as pltpu
from jax.experimental.pallas import tpu_sc as plsc

NC = 2
NS = 16
NTILES = NC * NS
CHUNK = 128
N_PAD = 10240
ROWS_PER_TILE = N_PAD // NTILES
N_GRAPHS = 64


@functools.lru_cache(maxsize=None)
def _seg_sum_call(d, nch):
    mesh = plsc.VectorSubcoreMesh(core_axis_name="c", subcore_axis_name="s")

    @functools.partial(
        pl.kernel,
        out_type=jax.ShapeDtypeStruct((NC, N_PAD, d), jnp.float32),
        mesh=mesh,
        compiler_params=pltpu.CompilerParams(use_tc_tiling_on_sc=False),
        scratch_types=[
            pltpu.VMEM((nch, CHUNK), jnp.int32),
            pltpu.VMEM((nch, CHUNK), jnp.int32),
            pltpu.VMEM((CHUNK, d), jnp.float32),
            pltpu.VMEM_SHARED((N_PAD, d), jnp.float32),
            pltpu.SemaphoreType.DMA,
        ],
    )
    def seg(table, srcs, dsts, zeros, out, src_v, dst_v, buf, accum, sem):
        c = lax.axis_index("c")
        s = lax.axis_index("s")
        wid = s * NC + c
        pltpu.sync_copy(srcs.at[wid], src_v)
        pltpu.sync_copy(dsts.at[wid], dst_v)
        r0 = s * ROWS_PER_TILE
        pltpu.sync_copy(zeros.at[pl.ds(r0, ROWS_PER_TILE)],
                        accum.at[pl.ds(r0, ROWS_PER_TILE)])
        plsc.subcore_barrier()

        @pl.loop(0, nch)
        def _chunk(j):
            pltpu.async_copy(table.at[src_v.at[j]], buf, sem).wait()
            pltpu.sync_copy(buf, accum.at[dst_v.at[j]], add=True)

        plsc.subcore_barrier()
        pltpu.sync_copy(accum.at[pl.ds(r0, ROWS_PER_TILE)],
                        out.at[c, pl.ds(r0, ROWS_PER_TILE)])

    return seg


def _seg_sum(table, srcs, dsts):
    d = table.shape[1]
    nch = srcs.shape[1]
    zeros = jnp.zeros((N_PAD, d), jnp.float32)
    return _seg_sum_call(d, nch)(table, srcs, dsts, zeros)


def _dot(a, b):
    return lax.dot_general(a, b, (((1,), (0,)), ((), ())),
                           preferred_element_type=jnp.float32,
                           precision=lax.Precision.HIGHEST)


def _mm(x, w):
    def body(x_ref, w_ref, o_ref):
        o_ref[...] = _dot(x_ref[...], w_ref[...])

    return pl.pallas_call(
        body,
        out_shape=jax.ShapeDtypeStruct((x.shape[0], w.shape[1]), jnp.float32),
    )(x, w)


def _combine(parts, n, xcat, wrel, brel, wroot, gamma, beta, relu):
    post = wrel is not None

    def body(p_ref, x_ref, *refs):
        if post:
            w_ref, b_ref, wr_ref, g_ref, be_ref, o_ref = refs
        else:
            b_ref, wr_ref, g_ref, be_ref, o_ref = refs
        agg = p_ref[0, :n, :] + p_ref[1, :n, :]
        if post:
            agg = _dot(agg, w_ref[...])
        pre = agg + b_ref[...] + _dot(x_ref[...], wr_ref[...])
        mu = jnp.mean(pre, axis=0, keepdims=True)
        var = jnp.mean((pre - mu) ** 2, axis=0, keepdims=True)
        h = (pre - mu) / jnp.sqrt(var + 1e-5) * g_ref[...] + be_ref[...]
        if relu:
            h = jnp.maximum(h, 0.0)
        o_ref[...] = h

    dout = wroot.shape[1]
    args = [parts, xcat]
    if post:
        args.append(wrel)
    args += [brel.reshape(1, -1), wroot, gamma.reshape(1, -1),
             beta.reshape(1, -1)]
    return pl.pallas_call(
        body,
        out_shape=jax.ShapeDtypeStruct((n, dout), jnp.float32),
    )(*args)


def _tail(h, batch2d, w, b):
    n, d = h.shape

    def body(h_ref, b_ref, w_ref, bias_ref, o_ref):
        oh = (b_ref[...] == lax.broadcasted_iota(jnp.int32, (n, N_GRAPHS), 1)
              ).astype(jnp.float32)
        sums = lax.dot_general(oh, h_ref[...], (((0,), (0,)), ((), ())),
                               preferred_element_type=jnp.float32,
                               precision=lax.Precision.HIGHEST)
        counts = lax.dot_general(oh, jnp.ones((n, 1), jnp.float32),
                                 (((0,), (0,)), ((), ())),
                                 preferred_element_type=jnp.float32,
                                 precision=lax.Precision.HIGHEST)
        pooled = sums / jnp.maximum(counts, 1.0)
        o_ref[...] = _dot(pooled, w_ref[...]) + bias_ref[...]

    return pl.pallas_call(
        body,
        out_shape=jax.ShapeDtypeStruct((N_GRAPHS, w.shape[1]), jnp.float32),
    )(h, batch2d, w, b.reshape(1, -1))


def kernel(x, edge_index, batch, params):
    n = x.shape[0]
    e = edge_index.shape[1]
    ept = -(-e // (NTILES * CHUNK)) * CHUNK
    epad = NTILES * ept - e
    nch = ept // CHUNK
    srcs = jnp.concatenate(
        [edge_index[0], jnp.zeros((epad,), jnp.int32)]).reshape(NTILES, nch, CHUNK)
    dsts = jnp.concatenate(
        [edge_index[1], jnp.full((epad,), n, jnp.int32)]).reshape(NTILES, nch, CHUNK)

    def gconv(p, bnp, xcat, relu):
        din, dout = p['Wrel'].shape
        if dout < din:
            parts = _seg_sum(_mm(xcat, p['Wrel']), srcs, dsts)
            return _combine(parts, n, xcat, None, p['brel'], p['Wroot'],
                            bnp['gamma'], bnp['beta'], relu)
        parts = _seg_sum(xcat, srcs, dsts)
        return _combine(parts, n, xcat, p['Wrel'], p['brel'], p['Wroot'],
                        bnp['gamma'], bnp['beta'], relu)

    h = gconv(params['init_conv'], params['init_bn'], x, False)
    for blk in params['blocks']:
        cat = h
        for lyr in blk['layers']:
            z = gconv(lyr['conv1'], lyr['bn1'], cat, True)
            z = gconv(lyr['conv2'], lyr['bn2'], z, True)
            cat = jnp.concatenate([cat, z], axis=1)
        h = gconv(blk['trans_conv'], blk['trans_bn'], cat, False)

    return _tail(h, batch.reshape(n, 1), params['proj']['W'], params['proj']['b'])
